# Initial kernel scaffold; baseline (speedup 1.0000x reference)
import jax
import jax.numpy as jnp
from jax.experimental import pallas as pl


def kernel(x):
    raise NotImplementedError("write your pallas kernel here")


# speedup vs baseline: 489.8640x; 489.8640x over previous
import functools

import jax
import jax.numpy as jnp
from jax import lax
from jax.experimental import pallas as pl
from jax.experimental.pallas import tpu as pltpu
from jax.experimental.pallas import tpu_sc as plsc

REGION_NUM = 8
NUM_CORES = 2
NUM_SUBCORES = 16
L = 16
NW = NUM_CORES * NUM_SUBCORES


def _make_sc_kernel(C, N):
    cpw = C // NW
    nv = N // L
    U1 = 8
    U2 = 4
    n1 = nv // U1
    n2 = nv // U2

    mesh = plsc.VectorSubcoreMesh(
        core_axis_name="c", subcore_axis_name="s",
        num_cores=NUM_CORES, num_subcores=NUM_SUBCORES)

    @functools.partial(
        pl.kernel,
        out_type=jax.ShapeDtypeStruct((C, N), jnp.float32),
        mesh=mesh,
        scratch_types=[
            pltpu.VMEM((N,), jnp.float32),
            pltpu.VMEM((L,), jnp.float32),
        ],
    )
    def k(x_hbm, rp_hbm, out_hbm, buf, rp_v):
        wid = lax.axis_index("s") * NUM_CORES + lax.axis_index("c")
        for s in range(cpw):
            ch = wid * cpw + s
            pltpu.sync_copy(x_hbm.at[ch], buf)
            pltpu.sync_copy(rp_hbm.at[ch], rp_v)

            def red_body(j, carry):
                vmn, vmx = carry
                base = j * (U1 * L)
                vs = [buf[pl.ds(base + u * L, L)] for u in range(U1)]
                lo = vs[0]
                hi = vs[0]
                for u in range(1, U1):
                    lo = jnp.minimum(lo, vs[u])
                    hi = jnp.maximum(hi, vs[u])
                return jnp.minimum(vmn, lo), jnp.maximum(vmx, hi)

            v0 = buf[pl.ds(0, L)]
            vmn, vmx = lax.fori_loop(0, n1, red_body, (v0, v0))

            ii = lax.iota(jnp.int32, L)
            for d in (8, 4, 2, 1):
                perm = ii ^ d
                vmn = jnp.minimum(vmn, vmn.at[perm].get(mode="promise_in_bounds"))
                vmx = jnp.maximum(vmx, vmx.at[perm].get(mode="promise_in_bounds"))
            mn_splat = vmn
            rng_splat = vmx - vmn

            t = rp_v[...] * rng_splat + mn_splat
            bs = []
            for i in range(REGION_NUM - 1):
                idx = jnp.full((L,), i, jnp.int32)
                bs.append(t.at[idx].get(mode="promise_in_bounds"))

            def map_body(j, carry):
                base = j * (U2 * L)
                for u in range(U2):
                    v = buf[pl.ds(base + u * L, L)]
                    o = mn_splat
                    for b in bs:
                        o = jnp.where(v >= b, b, o)
                    buf[pl.ds(base + u * L, L)] = o
                return carry

            lax.fori_loop(0, n2, map_body, 0)
            pltpu.sync_copy(buf, out_hbm.at[ch])

    return k


def kernel(x):
    B, c, H, W = x.shape
    C = B * c
    N = H * W
    xr = x.reshape(C, N)
    rp = jax.random.uniform(
        jax.random.key(42), (C * (REGION_NUM - 1),), dtype=jnp.float32
    ).reshape(C, REGION_NUM - 1)
    rp = jnp.sort(rp, axis=1)
    rp16 = jnp.concatenate(
        [rp, jnp.full((C, L - (REGION_NUM - 1)), 2.0, jnp.float32)], axis=1)
    out = _make_sc_kernel(C, N)(xr, rp16)
    return out.reshape(B, c, H, W)


# speedup vs baseline: 543.6298x; 1.1098x over previous
import functools

import jax
import jax.numpy as jnp
from jax import lax
from jax.experimental import pallas as pl
from jax.experimental.pallas import tpu as pltpu
from jax.experimental.pallas import tpu_sc as plsc

REGION_NUM = 8
NUM_CORES = 2
NUM_SUBCORES = 16
L = 16
NW = NUM_CORES * NUM_SUBCORES


def _make_sc_kernel(C, N):
    cpw = C // NW
    nv = N // L
    U1 = 8
    U2 = 8
    n1 = nv // U1
    n2 = nv // U2

    mesh = plsc.VectorSubcoreMesh(
        core_axis_name="c", subcore_axis_name="s",
        num_cores=NUM_CORES, num_subcores=NUM_SUBCORES)

    @functools.partial(
        pl.kernel,
        out_type=jax.ShapeDtypeStruct((C, N), jnp.float32),
        mesh=mesh,
        scratch_types=[
            pltpu.VMEM((N,), jnp.float32),
            pltpu.VMEM((N,), jnp.float32),
            pltpu.VMEM((cpw * L,), jnp.float32),
            pltpu.SemaphoreType.DMA,
            pltpu.SemaphoreType.DMA,
            pltpu.SemaphoreType.DMA,
            pltpu.SemaphoreType.DMA,
        ],
    )
    def k(x_hbm, rp_hbm, out_hbm, buf0, buf1, rp_buf, si0, si1, so0, so1):
        wid = lax.axis_index("s") * NUM_CORES + lax.axis_index("c")
        base_ch = wid * cpw
        bufs = (buf0, buf1)
        isems = (si0, si1)
        osems = (so0, so1)
        pltpu.sync_copy(rp_hbm.at[pl.ds(base_ch * L, cpw * L)], rp_buf)

        def in_copy(s):
            return pltpu.make_async_copy(
                x_hbm.at[base_ch + s], bufs[s % 2], isems[s % 2])

        def out_copy(s):
            return pltpu.make_async_copy(
                bufs[s % 2], out_hbm.at[base_ch + s], osems[s % 2])

        in_copy(0).start()
        for s in range(cpw):
            buf = bufs[s % 2]
            in_copy(s).wait()
            if s + 1 < cpw:
                if s >= 1:
                    out_copy(s - 1).wait()
                in_copy(s + 1).start()

            def red_body(j, carry):
                vmn, vmx = carry
                base = j * (U1 * L)
                vs = [buf[pl.ds(base + u * L, L)] for u in range(U1)]
                lo = vs[0]
                hi = vs[0]
                for u in range(1, U1):
                    lo = jnp.minimum(lo, vs[u])
                    hi = jnp.maximum(hi, vs[u])
                return jnp.minimum(vmn, lo), jnp.maximum(vmx, hi)

            v0 = buf[pl.ds(0, L)]
            vmn, vmx = lax.fori_loop(0, n1, red_body, (v0, v0))

            ii = lax.iota(jnp.int32, L)
            for d in (8, 4, 2, 1):
                perm = ii ^ d
                vmn = jnp.minimum(vmn, vmn.at[perm].get(mode="promise_in_bounds"))
                vmx = jnp.maximum(vmx, vmx.at[perm].get(mode="promise_in_bounds"))
            mn_splat = vmn
            rng_splat = vmx - vmn

            t = rp_buf[pl.ds(s * L, L)] * rng_splat + mn_splat
            bs = []
            for i in range(REGION_NUM - 1):
                idx = jnp.full((L,), i, jnp.int32)
                bs.append(t.at[idx].get(mode="promise_in_bounds"))

            def map_body(j, carry):
                base = j * (U2 * L)
                for u in range(U2):
                    v = buf[pl.ds(base + u * L, L)]
                    o = mn_splat
                    for b in bs:
                        o = jnp.where(v >= b, b, o)
                    buf[pl.ds(base + u * L, L)] = o
                return carry

            lax.fori_loop(0, n2, map_body, 0)
            out_copy(s).start()

        out_copy(cpw - 2).wait()
        out_copy(cpw - 1).wait()

    return k


def kernel(x):
    B, c, H, W = x.shape
    C = B * c
    N = H * W
    xr = x.reshape(C, N)
    rp = jax.random.uniform(
        jax.random.key(42), (C * (REGION_NUM - 1),), dtype=jnp.float32
    ).reshape(C, REGION_NUM - 1)
    rp = jnp.sort(rp, axis=1)
    rp16 = jnp.concatenate(
        [rp, jnp.full((C, L - (REGION_NUM - 1)), 2.0, jnp.float32)],
        axis=1).reshape(C * L)
    out = _make_sc_kernel(C, N)(xr, rp16)
    return out.reshape(B, c, H, W)


# speedup vs baseline: 997.8109x; 1.8355x over previous
import functools

import jax
import jax.numpy as jnp
from jax import lax
from jax.experimental import pallas as pl
from jax.experimental.pallas import tpu as pltpu
from jax.experimental.pallas import tpu_sc as plsc

REGION_NUM = 8
NUM_CORES = 2
NUM_SUBCORES = 16
L = 16
NW = NUM_CORES * NUM_SUBCORES


def _make_sc_kernel(B, c, H, W):
    C = B * c
    cpw = C // NW
    vpr = W // L

    mesh = plsc.VectorSubcoreMesh(
        core_axis_name="c", subcore_axis_name="s",
        num_cores=NUM_CORES, num_subcores=NUM_SUBCORES)

    @functools.partial(
        pl.kernel,
        out_type=jax.ShapeDtypeStruct((B, c, H, W), jnp.float32),
        mesh=mesh,
        scratch_types=[
            pltpu.VMEM((H, W), jnp.float32),
            pltpu.VMEM((H, W), jnp.float32),
            pltpu.VMEM((cpw * L,), jnp.float32),
            pltpu.SemaphoreType.DMA,
            pltpu.SemaphoreType.DMA,
            pltpu.SemaphoreType.DMA,
            pltpu.SemaphoreType.DMA,
        ],
    )
    def k(x_hbm, rp_hbm, out_hbm, buf0, buf1, rp_buf, si0, si1, so0, so1):
        wid = lax.axis_index("s") * NUM_CORES + lax.axis_index("c")
        base_ch = wid * cpw
        bufs = (buf0, buf1)
        isems = (si0, si1)
        osems = (so0, so1)
        pltpu.sync_copy(rp_hbm.at[pl.ds(base_ch * L, cpw * L)], rp_buf)

        def ch_idx(s):
            ch = base_ch + s
            return ch // c, ch % c

        def in_copy(s):
            b, cc = ch_idx(s)
            return pltpu.make_async_copy(
                x_hbm.at[b, cc], bufs[s % 2], isems[s % 2])

        def out_copy(s):
            b, cc = ch_idx(s)
            return pltpu.make_async_copy(
                bufs[s % 2], out_hbm.at[b, cc], osems[s % 2])

        in_copy(0).start()
        for s in range(cpw):
            buf = bufs[s % 2]
            in_copy(s).wait()
            if s + 1 < cpw:
                if s >= 1:
                    out_copy(s - 1).wait()
                in_copy(s + 1).start()

            def red_body(r, carry):
                vmn, vmx = carry
                vs = [buf[r, pl.ds(u * L, L)] for u in range(vpr)]
                lo = vs[0]
                hi = vs[0]
                for u in range(1, vpr):
                    lo = jnp.minimum(lo, vs[u])
                    hi = jnp.maximum(hi, vs[u])
                return jnp.minimum(vmn, lo), jnp.maximum(vmx, hi)

            v0 = buf[0, pl.ds(0, L)]
            vmn, vmx = lax.fori_loop(0, H, red_body, (v0, v0))

            ii = lax.iota(jnp.int32, L)
            for d in (8, 4, 2, 1):
                perm = ii ^ d
                vmn = jnp.minimum(vmn, vmn.at[perm].get(mode="promise_in_bounds"))
                vmx = jnp.maximum(vmx, vmx.at[perm].get(mode="promise_in_bounds"))
            mn_splat = vmn
            rng_splat = vmx - vmn

            t = rp_buf[pl.ds(s * L, L)] * rng_splat + mn_splat
            bs = []
            for i in range(REGION_NUM - 1):
                idx = jnp.full((L,), i, jnp.int32)
                bs.append(t.at[idx].get(mode="promise_in_bounds"))

            def map_body(r, carry):
                for u in range(vpr):
                    v = buf[r, pl.ds(u * L, L)]
                    o = mn_splat
                    for b in bs:
                        o = jnp.where(v >= b, b, o)
                    buf[r, pl.ds(u * L, L)] = o
                return carry

            lax.fori_loop(0, H, map_body, 0)
            out_copy(s).start()

        out_copy(cpw - 2).wait()
        out_copy(cpw - 1).wait()

    return k


def kernel(x):
    B, c, H, W = x.shape
    C = B * c
    rp = jax.random.uniform(
        jax.random.key(42), (C * (REGION_NUM - 1),), dtype=jnp.float32
    ).reshape(C, REGION_NUM - 1)
    rp = jnp.sort(rp, axis=1)
    rp16 = jnp.concatenate(
        [rp, jnp.full((C, L - (REGION_NUM - 1)), 2.0, jnp.float32)],
        axis=1).reshape(C * L)
    return _make_sc_kernel(B, c, H, W)(x, rp16)


# speedup vs baseline: 1119.8906x; 1.1223x over previous
import functools

import jax
import jax.numpy as jnp
from jax import lax
from jax.experimental import pallas as pl
from jax.experimental.pallas import tpu as pltpu
from jax.experimental.pallas import tpu_sc as plsc

REGION_NUM = 8
NUM_CORES = 2
NUM_SUBCORES = 16
L = 16
NW = NUM_CORES * NUM_SUBCORES


def _make_sc_kernel(B, c, H, W):
    C = B * c
    cpw = C // NW
    vpr = W // L

    mesh = plsc.VectorSubcoreMesh(
        core_axis_name="c", subcore_axis_name="s",
        num_cores=NUM_CORES, num_subcores=NUM_SUBCORES)

    @functools.partial(
        pl.kernel,
        out_type=jax.ShapeDtypeStruct((B, c, H, W), jnp.float32),
        mesh=mesh,
        scratch_types=[
            pltpu.VMEM((H, W), jnp.float32),
            pltpu.VMEM((H, W), jnp.float32),
            pltpu.VMEM((cpw * L,), jnp.float32),
            pltpu.SemaphoreType.DMA,
            pltpu.SemaphoreType.DMA,
            pltpu.SemaphoreType.DMA,
            pltpu.SemaphoreType.DMA,
        ],
    )
    def k(x_hbm, rp_hbm, out_hbm, buf0, buf1, rp_buf, si0, si1, so0, so1):
        wid = lax.axis_index("s") * NUM_CORES + lax.axis_index("c")
        base_ch = wid * cpw
        bufs = (buf0, buf1)
        isems = (si0, si1)
        osems = (so0, so1)
        pltpu.sync_copy(rp_hbm.at[pl.ds(base_ch * L, cpw * L)], rp_buf)

        def ch_idx(s):
            ch = base_ch + s
            return ch // c, ch % c

        def in_copy(s):
            b, cc = ch_idx(s)
            return pltpu.make_async_copy(
                x_hbm.at[b, cc], bufs[s % 2], isems[s % 2])

        def out_copy(s):
            b, cc = ch_idx(s)
            return pltpu.make_async_copy(
                bufs[s % 2], out_hbm.at[b, cc], osems[s % 2])

        in_copy(0).start()
        for s in range(cpw):
            buf = bufs[s % 2]
            in_copy(s).wait()
            if s + 1 < cpw:
                if s >= 1:
                    out_copy(s - 1).wait()
                in_copy(s + 1).start()

            def red_body(r, carry):
                vmn, vmx = carry
                vs = [buf[r, pl.ds(u * L, L)] for u in range(vpr)]
                lo = vs[0]
                hi = vs[0]
                for u in range(1, vpr):
                    lo = jnp.minimum(lo, vs[u])
                    hi = jnp.maximum(hi, vs[u])
                return jnp.minimum(vmn, lo), jnp.maximum(vmx, hi)

            v0 = buf[0, pl.ds(0, L)]
            vmn, vmx = lax.fori_loop(0, H, red_body, (v0, v0))

            ii = lax.iota(jnp.int32, L)
            for d in (8, 4, 2, 1):
                perm = ii ^ d
                vmn = jnp.minimum(vmn, vmn.at[perm].get(mode="promise_in_bounds"))
                vmx = jnp.maximum(vmx, vmx.at[perm].get(mode="promise_in_bounds"))
            mn_splat = vmn
            rng_splat = vmx - vmn

            t = rp_buf[pl.ds(s * L, L)] * rng_splat + mn_splat

            im1 = jnp.maximum(ii - 1, jnp.full((L,), 0, jnp.int32))
            tshift = t.at[im1].get(mode="promise_in_bounds")
            tbl = jnp.where(ii == 0, mn_splat, tshift)
            p4s = tbl.at[jnp.full((L,), 4, jnp.int32)].get(
                mode="promise_in_bounds")
            k4 = jnp.full((L,), 4, jnp.int32)
            k0 = jnp.full((L,), 0, jnp.int32)
            k6 = jnp.full((L,), 6, jnp.int32)
            k2 = jnp.full((L,), 2, jnp.int32)
            one = jnp.full((L,), 1, jnp.int32)

            def map_body(r, carry):
                for u in range(vpr):
                    v = buf[r, pl.ds(u * L, L)]
                    m1 = v >= p4s
                    k = jnp.where(m1, k4, k0)
                    c2 = jnp.where(m1, k6, k2)
                    p2 = tbl.at[c2].get(mode="promise_in_bounds")
                    k = jnp.where(v >= p2, c2, k)
                    c3 = k + one
                    p3 = tbl.at[c3].get(mode="promise_in_bounds")
                    k = jnp.where(v >= p3, c3, k)
                    buf[r, pl.ds(u * L, L)] = tbl.at[k].get(
                        mode="promise_in_bounds")
                return carry

            lax.fori_loop(0, H, map_body, 0)
            out_copy(s).start()

        out_copy(cpw - 2).wait()
        out_copy(cpw - 1).wait()

    return k


def kernel(x):
    B, c, H, W = x.shape
    C = B * c
    rp = jax.random.uniform(
        jax.random.key(42), (C * (REGION_NUM - 1),), dtype=jnp.float32
    ).reshape(C, REGION_NUM - 1)
    rp = jnp.sort(rp, axis=1)
    rp16 = jnp.concatenate(
        [rp, jnp.full((C, L - (REGION_NUM - 1)), 2.0, jnp.float32)],
        axis=1).reshape(C * L)
    return _make_sc_kernel(B, c, H, W)(x, rp16)
